# baseline (speedup 1.0000x reference)
import functools

import jax
import jax.numpy as jnp
from jax import lax
from jax.experimental import pallas as pl
from jax.experimental.pallas import tpu as pltpu
from jax.experimental.pallas import tpu_sc as plsc

_B, _K, _C, _W, _H = 256, 133, 256, 768, 1024
_KTOP = 10
_TB = 8
_NB = _B // _TB
_NC, _NS, _L = 2, 16, 16
_NW = _NC * _NS
_KCH = 9


def _scores_body(hx_ref, hy_ref, out_ref):
    cols = []
    for tb in range(_TB):
        hx = hx_ref[tb]
        hy = hy_ref[tb]
        sx = jnp.sum(jnp.exp(hx - jnp.max(hx, axis=-1, keepdims=True)),
                     axis=-1, keepdims=True)
        sy = jnp.sum(jnp.exp(hy - jnp.max(hy, axis=-1, keepdims=True)),
                     axis=-1, keepdims=True)
        cols.append(1.0 - 0.5 * (1.0 / sx + 1.0 / sy))
    out_ref[0] = jnp.concatenate(cols, axis=1)


def _sc_body(scores_hbm, f_hbm, canon_hbm, outf_hbm, mask_hbm,
             slab_v, fbuf_v, rows_v, idx_v, mbuf_v, sem):
    wid = lax.axis_index("s") * _NC + lax.axis_index("c")
    lane = lax.iota(jnp.int32, _L)

    pltpu.sync_copy(scores_hbm.at[wid], slab_v)

    for j in range(_TB):
        cur = []
        col = jnp.full((_L,), j, jnp.int32)
        for i in range(_KCH):
            ridx = jnp.minimum(lane + 16 * i, _K - 1)
            v = plsc.load_gather(slab_v, [ridx, col])
            v = jnp.where(lane + 16 * i < _K, v, -1.0)
            cur.append(v)

        msel = [jnp.zeros((_L,), jnp.bool_) for _ in range(_KCH)]
        idxvec = jnp.zeros((_L,), jnp.int32)
        idx = jnp.int32(0)
        for t in range(_KTOP):
            mvec = cur[0]
            for i in range(1, _KCH):
                mvec = jnp.maximum(mvec, cur[i])
            m = jnp.max(mvec)
            cand = jnp.full((_L,), 10000, jnp.int32)
            for i in range(_KCH):
                cand = jnp.minimum(cand,
                                   jnp.where(cur[i] == m, lane + 16 * i, 10000))
            idx = jnp.min(cand)
            idxvec = jnp.where(lane == t, idx, idxvec)
            for i in range(_KCH):
                hit = (lane + 16 * i) == idx
                msel[i] = msel[i] | hit
                cur[i] = jnp.where(hit, -2.0, cur[i])
        idxvec = jnp.where(lane < _KTOP, idxvec, idx)
        idx_v[...] = idxvec

        for i in range(_KCH):
            mbuf_v[pl.ds(16 * i, 16)] = msel[i].astype(jnp.int32)
        pltpu.sync_copy(mbuf_v, mask_hbm.at[wid * _TB + j])

        pltpu.sync_copy(f_hbm.at[wid * _TB + j], fbuf_v)
        pltpu.async_copy(canon_hbm.at[idx_v], rows_v, sem).wait()
        pltpu.sync_copy(fbuf_v, outf_hbm.at[wid * _TB + j])
        pltpu.async_copy(rows_v, outf_hbm.at[wid * _TB + j].at[idx_v],
                         sem).wait()


_sc_call = functools.partial(
    pl.kernel,
    out_type=[
        jax.ShapeDtypeStruct((_B, _K, _C), jnp.float32),
        jax.ShapeDtypeStruct((_B, _KCH * _L), jnp.int32),
    ],
    mesh=plsc.VectorSubcoreMesh(core_axis_name="c", subcore_axis_name="s"),
    compiler_params=pltpu.CompilerParams(needs_layout_passes=False,
                                         use_tc_tiling_on_sc=True),
    scratch_types=[
        pltpu.VMEM((_K, _TB), jnp.float32),
        pltpu.VMEM((_K, _C), jnp.float32),
        pltpu.VMEM((_L, _C), jnp.float32),
        pltpu.VMEM((_L,), jnp.int32),
        pltpu.VMEM((_KCH * _L,), jnp.int32),
        pltpu.SemaphoreType.DMA,
    ],
)


def kernel(f_kpts, h_initial_x, h_initial_y, canonical_table):
    scores_t = pl.pallas_call(
        _scores_body,
        grid=(_NB,),
        in_specs=[
            pl.BlockSpec((_TB, _K, _W), lambda i: (i, 0, 0)),
            pl.BlockSpec((_TB, _K, _H), lambda i: (i, 0, 0)),
        ],
        out_specs=pl.BlockSpec((1, _K, _TB), lambda i: (i, 0, 0)),
        out_shape=jax.ShapeDtypeStruct((_NB, _K, _TB), jnp.float32),
        compiler_params=pltpu.CompilerParams(
            dimension_semantics=("parallel",),
        ),
    )(h_initial_x, h_initial_y)

    out_f, mask_rows = _sc_call(_sc_body)(scores_t, f_kpts, canonical_table)
    return out_f, (mask_rows[:, :_K] != 0)


# speedup vs baseline: 1.0262x; 1.0262x over previous
import functools

import jax
import jax.numpy as jnp
from jax import lax
from jax.experimental import pallas as pl
from jax.experimental.pallas import tpu as pltpu
from jax.experimental.pallas import tpu_sc as plsc

_B, _K, _C, _W, _H = 256, 133, 256, 768, 1024
_KTOP = 10
_TB = 8
_NB = _B // _TB
_NC, _NS, _L = 2, 16, 16
_NW = _NC * _NS
_KCH = 9


def _scores_body(hx_ref, hy_ref, out_ref):
    cols = []
    for tb in range(_TB):
        hx = hx_ref[tb]
        hy = hy_ref[tb]
        sx = jnp.sum(jnp.exp(hx - jnp.max(hx, axis=-1, keepdims=True)),
                     axis=-1, keepdims=True)
        sy = jnp.sum(jnp.exp(hy - jnp.max(hy, axis=-1, keepdims=True)),
                     axis=-1, keepdims=True)
        cols.append(1.0 - 0.5 * (1.0 / sx + 1.0 / sy))
    out_ref[0] = jnp.concatenate(cols, axis=1)


def _topk_col(slab_v, j, lane):
    cur = []
    col = jnp.full((_L,), j, jnp.int32)
    for i in range(_KCH):
        ridx = jnp.minimum(lane + 16 * i, _K - 1)
        v = plsc.load_gather(slab_v, [ridx, col])
        v = jnp.where(lane + 16 * i < _K, v, -1.0)
        cur.append(v)
    msel = [jnp.zeros((_L,), jnp.bool_) for _ in range(_KCH)]
    idxs = []
    for _ in range(_KTOP):
        mvec = cur[0]
        for i in range(1, _KCH):
            mvec = jnp.maximum(mvec, cur[i])
        m = jnp.max(mvec)
        cand = jnp.full((_L,), 10000, jnp.int32)
        for i in range(_KCH):
            cand = jnp.minimum(cand,
                               jnp.where(cur[i] == m, lane + 16 * i, 10000))
        idx = jnp.min(cand)
        idxs.append(idx)
        for i in range(_KCH):
            hit = (lane + 16 * i) == idx
            msel[i] = msel[i] | hit
            cur[i] = jnp.where(hit, -2.0, cur[i])
    return msel, idxs


def _sc_body(scores_hbm, f_hbm, canon_hbm, outf_hbm, mask_hbm,
             slab_v, rows_v, idx_v, fbuf, mbuf, semin, semout, semm, semc):
    wid = lax.axis_index("s") * _NC + lax.axis_index("c")
    lane = lax.iota(jnp.int32, _L)
    b0 = wid * _TB

    pltpu.sync_copy(scores_hbm.at[wid], slab_v)
    hin = {0: pltpu.async_copy(f_hbm.at[b0], fbuf[0], semin[0])}
    hout, hm = {}, {}

    for j in range(_TB):
        p = j % 2
        q = 1 - p
        if j + 1 < _TB:
            if j - 1 >= 0:
                hout[j - 1].wait()
            hin[j + 1] = pltpu.async_copy(f_hbm.at[b0 + j + 1], fbuf[q],
                                          semin[q])
        msel, idxs = _topk_col(slab_v, j, lane)

        idxvec = jnp.zeros((_L,), jnp.int32)
        for t in range(_KTOP):
            idxvec = jnp.where(lane == t, idxs[t], idxvec)
        idxvec = jnp.where(lane < _KTOP, idxvec, idxs[-1])
        idx_v[...] = idxvec
        hg = pltpu.async_copy(canon_hbm.at[idx_v], rows_v, semc)
        hin[j].wait()
        hg.wait()
        for t in range(_KTOP):
            row = jnp.full((_L,), idxs[t], jnp.int32)
            for c in range(_C // _L):
                cidx = lane + 16 * c
                val = rows_v[t, pl.ds(16 * c, 16)]
                plsc.store_scatter(fbuf[p], [row, cidx], val)
        hout[j] = pltpu.async_copy(fbuf[p], outf_hbm.at[b0 + j], semout[p])

        for i in range(_KCH):
            mbuf[p][pl.ds(16 * i, 16)] = msel[i].astype(jnp.int32)
        if j - 2 >= 0:
            hm[j - 2].wait()
        hm[j] = pltpu.async_copy(mbuf[p], mask_hbm.at[b0 + j], semm[p])

    for j in (_TB - 2, _TB - 1):
        hout[j].wait()
        hm[j].wait()


_sc_call = functools.partial(
    pl.kernel,
    out_type=[
        jax.ShapeDtypeStruct((_B, _K, _C), jnp.float32),
        jax.ShapeDtypeStruct((_B, _KCH * _L), jnp.int32),
    ],
    mesh=plsc.VectorSubcoreMesh(core_axis_name="c", subcore_axis_name="s"),
    compiler_params=pltpu.CompilerParams(needs_layout_passes=False,
                                         use_tc_tiling_on_sc=True),
    scratch_types=[
        pltpu.VMEM((_K, _TB), jnp.float32),
        pltpu.VMEM((_L, _C), jnp.float32),
        pltpu.VMEM((_L,), jnp.int32),
        [pltpu.VMEM((_K, _C), jnp.float32)] * 2,
        [pltpu.VMEM((_KCH * _L,), jnp.int32)] * 2,
        [pltpu.SemaphoreType.DMA] * 2,
        [pltpu.SemaphoreType.DMA] * 2,
        [pltpu.SemaphoreType.DMA] * 2,
        pltpu.SemaphoreType.DMA,
    ],
)


def kernel(f_kpts, h_initial_x, h_initial_y, canonical_table):
    scores_t = pl.pallas_call(
        _scores_body,
        grid=(_NB,),
        in_specs=[
            pl.BlockSpec((_TB, _K, _W), lambda i: (i, 0, 0)),
            pl.BlockSpec((_TB, _K, _H), lambda i: (i, 0, 0)),
        ],
        out_specs=pl.BlockSpec((1, _K, _TB), lambda i: (i, 0, 0)),
        out_shape=jax.ShapeDtypeStruct((_NB, _K, _TB), jnp.float32),
        compiler_params=pltpu.CompilerParams(
            dimension_semantics=("parallel",),
        ),
    )(h_initial_x, h_initial_y)

    out_f, mask_rows = _sc_call(_sc_body)(scores_t, f_kpts, canonical_table)
    return out_f, (mask_rows[:, :_K] != 0)


# speedup vs baseline: 1.0534x; 1.0266x over previous
import functools

import jax
import jax.numpy as jnp
from jax import lax
from jax.experimental import pallas as pl
from jax.experimental.pallas import tpu as pltpu
from jax.experimental.pallas import tpu_sc as plsc

_B, _K, _C, _W, _H = 256, 133, 256, 768, 1024
_KTOP = 10
_TB = 8
_NB = _B // _TB
_NB1 = 8
_B1 = _NB1 * _TB
_NC, _NS, _L = 2, 16, 16
_NW = _NC * _NS
_BPW = _B1 // _NW
_KCH = 9


def _row_scores(hx_ref, hy_ref):
    cols = []
    for tb in range(_TB):
        hx = hx_ref[tb]
        hy = hy_ref[tb]
        sx = jnp.sum(jnp.exp(hx - jnp.max(hx, axis=-1, keepdims=True)),
                     axis=-1, keepdims=True)
        sy = jnp.sum(jnp.exp(hy - jnp.max(hy, axis=-1, keepdims=True)),
                     axis=-1, keepdims=True)
        cols.append(1.0 - 0.5 * (1.0 / sx + 1.0 / sy))
    return jnp.concatenate(cols, axis=1)


def _scores_body(hx_ref, hy_ref, out_ref):
    out_ref[0] = _row_scores(hx_ref, hy_ref)


def _vector_topk(score):
    iota = jax.lax.broadcasted_iota(jnp.int32, (_K, _TB), 0)
    mask = jnp.zeros((_K, _TB), dtype=jnp.bool_)
    for _ in range(_KTOP):
        cur = jnp.where(mask, -1.0, score)
        m = jnp.max(cur, axis=0, keepdims=True)
        idx = jnp.min(jnp.where(cur == m, iota, _K), axis=0, keepdims=True)
        mask = mask | (iota == idx)
    return mask


def _tail_body(base_ref, hx_ref, hy_ref, f_ref, canon_ref,
               out_f_ref, out_m_ref):
    del base_ref
    mask = _vector_topk(_row_scores(hx_ref, hy_ref))
    out_m_ref[0] = mask.astype(jnp.int32)
    canon = canon_ref[...]
    for tb in range(_TB):
        out_f_ref[tb] = jnp.where(mask[:, tb:tb + 1], canon, f_ref[tb])


def _topk_col(slab_v, j, lane):
    cur = []
    col = jnp.full((_L,), j, jnp.int32)
    for i in range(_KCH):
        ridx = jnp.minimum(lane + 16 * i, _K - 1)
        v = plsc.load_gather(slab_v, [ridx, col])
        v = jnp.where(lane + 16 * i < _K, v, -1.0)
        cur.append(v)
    msel = [jnp.zeros((_L,), jnp.bool_) for _ in range(_KCH)]
    idxs = []
    for _ in range(_KTOP):
        mvec = cur[0]
        for i in range(1, _KCH):
            mvec = jnp.maximum(mvec, cur[i])
        m = jnp.max(mvec)
        cand = jnp.full((_L,), 10000, jnp.int32)
        for i in range(_KCH):
            cand = jnp.minimum(cand,
                               jnp.where(cur[i] == m, lane + 16 * i, 10000))
        idx = jnp.min(cand)
        idxs.append(idx)
        for i in range(_KCH):
            hit = (lane + 16 * i) == idx
            msel[i] = msel[i] | hit
            cur[i] = jnp.where(hit, -2.0, cur[i])
    return msel, idxs


def _sc_body(scores_hbm, f_hbm, canon_hbm, outf_hbm, mask_hbm,
             slab_v, rows_v, idx_v, fbuf, mbuf, semin, semout, semm, semc):
    wid = lax.axis_index("s") * _NC + lax.axis_index("c")
    lane = lax.iota(jnp.int32, _L)
    b0 = wid * _BPW

    hin = {0: pltpu.async_copy(f_hbm.at[b0], fbuf[0], semin[0])}
    hout, hm = {}, {}

    for j in range(_BPW):
        b = b0 + j
        p = j % 2
        q = 1 - p
        if j + 1 < _BPW:
            if j - 1 >= 0:
                hout[j - 1].wait()
            hin[j + 1] = pltpu.async_copy(f_hbm.at[b + 1], fbuf[q], semin[q])
        pltpu.sync_copy(scores_hbm.at[b // _TB], slab_v)
        msel, idxs = _topk_col(slab_v, b % _TB, lane)

        idxvec = jnp.zeros((_L,), jnp.int32)
        for t in range(_KTOP):
            idxvec = jnp.where(lane == t, idxs[t], idxvec)
        idxvec = jnp.where(lane < _KTOP, idxvec, idxs[-1])
        idx_v[...] = idxvec
        hg = pltpu.async_copy(canon_hbm.at[idx_v], rows_v, semc)
        hin[j].wait()
        hg.wait()
        for t in range(_KTOP):
            row = jnp.full((_L,), idxs[t], jnp.int32)
            for c in range(_C // _L):
                cidx = lane + 16 * c
                val = rows_v[t, pl.ds(16 * c, 16)]
                plsc.store_scatter(fbuf[p], [row, cidx], val)
        hout[j] = pltpu.async_copy(fbuf[p], outf_hbm.at[b], semout[p])

        for i in range(_KCH):
            mbuf[p][pl.ds(16 * i, 16)] = msel[i].astype(jnp.int32)
        if j - 2 >= 0:
            hm[j - 2].wait()
        hm[j] = pltpu.async_copy(mbuf[p], mask_hbm.at[b], semm[p])

    for j in range(max(0, _BPW - 2), _BPW):
        hout[j].wait()
        hm[j].wait()


_sc_call = functools.partial(
    pl.kernel,
    out_type=[
        jax.ShapeDtypeStruct((_B, _K, _C), jnp.float32),
        jax.ShapeDtypeStruct((_B1, _KCH * _L), jnp.int32),
    ],
    mesh=plsc.VectorSubcoreMesh(core_axis_name="c", subcore_axis_name="s"),
    compiler_params=pltpu.CompilerParams(needs_layout_passes=False,
                                         use_tc_tiling_on_sc=True),
    scratch_types=[
        pltpu.VMEM((_K, _TB), jnp.float32),
        pltpu.VMEM((_L, _C), jnp.float32),
        pltpu.VMEM((_L,), jnp.int32),
        [pltpu.VMEM((_K, _C), jnp.float32)] * 2,
        [pltpu.VMEM((_KCH * _L,), jnp.int32)] * 2,
        [pltpu.SemaphoreType.DMA] * 2,
        [pltpu.SemaphoreType.DMA] * 2,
        [pltpu.SemaphoreType.DMA] * 2,
        pltpu.SemaphoreType.DMA,
    ],
)


def kernel(f_kpts, h_initial_x, h_initial_y, canonical_table):
    scores1 = pl.pallas_call(
        _scores_body,
        grid=(_NB1,),
        in_specs=[
            pl.BlockSpec((_TB, _K, _W), lambda i: (i, 0, 0)),
            pl.BlockSpec((_TB, _K, _H), lambda i: (i, 0, 0)),
        ],
        out_specs=pl.BlockSpec((1, _K, _TB), lambda i: (i, 0, 0)),
        out_shape=jax.ShapeDtypeStruct((_NB1, _K, _TB), jnp.float32),
        compiler_params=pltpu.CompilerParams(
            dimension_semantics=("parallel",),
        ),
    )(h_initial_x, h_initial_y)

    out_base, mask1 = _sc_call(_sc_body)(scores1, f_kpts, canonical_table)

    out_f, mask_t2 = pl.pallas_call(
        _tail_body,
        grid=(_NB - _NB1,),
        in_specs=[
            pl.BlockSpec((1, _K, _C), lambda i: (0, 0, 0)),
            pl.BlockSpec((_TB, _K, _W), lambda i: (_NB1 + i, 0, 0)),
            pl.BlockSpec((_TB, _K, _H), lambda i: (_NB1 + i, 0, 0)),
            pl.BlockSpec((_TB, _K, _C), lambda i: (_NB1 + i, 0, 0)),
            pl.BlockSpec((_K, _C), lambda i: (0, 0)),
        ],
        out_specs=[
            pl.BlockSpec((_TB, _K, _C), lambda i: (_NB1 + i, 0, 0)),
            pl.BlockSpec((1, _K, _TB), lambda i: (i, 0, 0)),
        ],
        out_shape=[
            jax.ShapeDtypeStruct((_B, _K, _C), jnp.float32),
            jax.ShapeDtypeStruct((_NB - _NB1, _K, _TB), jnp.int32),
        ],
        input_output_aliases={0: 0},
        compiler_params=pltpu.CompilerParams(
            dimension_semantics=("arbitrary",),
        ),
    )(out_base, h_initial_x, h_initial_y, f_kpts, canonical_table)

    m1 = mask1[:, :_K] != 0
    m2 = jnp.transpose(mask_t2, (0, 2, 1)).reshape(_B - _B1, _K) != 0
    return out_f, jnp.concatenate([m1, m2], axis=0)
